# Initial kernel scaffold; baseline (speedup 1.0000x reference)
import jax
import jax.numpy as jnp
from jax.experimental import pallas as pl


def kernel(hidden_states, cos, sin, Wq, Wk, Wv):
    raise NotImplementedError("write your pallas kernel here")


# speedup vs baseline: 1.1173x; 1.1173x over previous
import jax
import jax.numpy as jnp
from jax.experimental import pallas as pl

HIDDEN = 1024
NQ = 16
NKV = 4
HD = 64
S = 1024
GROUP = NQ // NKV


def _rope(x, cos, sin):
    x1 = x[:, : HD // 2]
    x2 = x[:, HD // 2:]
    rot = jnp.concatenate([-x2, x1], axis=1)
    return x * cos + rot * sin


def _fused_kernel(h_ref, wq_ref, wk_ref, wv_ref, cos_ref, sin_ref, o_ref):
    scaling = HD ** (-0.5)
    h = h_ref[...]
    dn = (((1,), (1,)), ((), ()))
    q_lin = jax.lax.dot_general(h, wq_ref[...], dn,
                                preferred_element_type=jnp.float32)
    k_lin = jax.lax.dot_general(h, wk_ref[...], dn,
                                preferred_element_type=jnp.float32)
    v = jax.lax.dot_general(h, wv_ref[...], dn,
                            preferred_element_type=jnp.float32)
    cos = cos_ref[...]
    sin = sin_ref[...]
    q = _rope(q_lin, cos, sin) * scaling
    k = _rope(k_lin, cos, sin)
    scores = jax.lax.dot_general(q, k, dn,
                                 preferred_element_type=jnp.float32)
    rows = jax.lax.broadcasted_iota(jnp.int32, (S, S), 0)
    cols = jax.lax.broadcasted_iota(jnp.int32, (S, S), 1)
    scores = jnp.where(rows >= cols, scores, -1e30)
    m = jnp.max(scores, axis=1, keepdims=True)
    e = jnp.exp(scores - m)
    p = e / jnp.sum(e, axis=1, keepdims=True)
    o_ref[0] = jnp.dot(p, v, preferred_element_type=jnp.float32)


def kernel(hidden_states, cos, sin, Wq, Wk, Wv):
    h2d = hidden_states[0]
    cos2d = cos[0]
    sin2d = sin[0]

    out = pl.pallas_call(
        _fused_kernel,
        grid=(NQ,),
        in_specs=[
            pl.BlockSpec((S, HIDDEN), lambda h: (0, 0)),
            pl.BlockSpec((HD, HIDDEN), lambda h: (h, 0)),
            pl.BlockSpec((HD, HIDDEN), lambda h: (h // GROUP, 0)),
            pl.BlockSpec((HD, HIDDEN), lambda h: (h // GROUP, 0)),
            pl.BlockSpec((S, HD), lambda h: (0, 0)),
            pl.BlockSpec((S, HD), lambda h: (0, 0)),
        ],
        out_specs=pl.BlockSpec((1, S, HD), lambda h: (h, 0, 0)),
        out_shape=jax.ShapeDtypeStruct((NQ, S, HD), jnp.float32),
    )(h2d, Wq, Wk, Wv, cos2d, sin2d)
    return out


# speedup vs baseline: 1.2685x; 1.1354x over previous
import jax
import jax.numpy as jnp
from jax.experimental import pallas as pl
from jax.experimental.pallas import tpu as pltpu

HIDDEN = 1024
NQ = 16
NKV = 4
HD = 64
S = 1024
GROUP = NQ // NKV
BQ = 256


def _rope(x, cos, sin):
    x1 = x[:, : HD // 2]
    x2 = x[:, HD // 2:]
    rot = jnp.concatenate([-x2, x1], axis=1)
    return x * cos + rot * sin


_DN = (((1,), (1,)), ((), ()))


def _fused_kernel(h_ref, wq_ref, wk_ref, wv_ref, cos_ref, sin_ref, o_ref,
                  k_scr, v_scr):
    scaling = HD ** (-0.5)
    head = pl.program_id(0)
    cos = cos_ref[...]
    sin = sin_ref[...]

    @pl.when(head % GROUP == 0)
    def _project_kv():
        h = h_ref[...]
        k_lin = jax.lax.dot_general(h, wk_ref[...], _DN,
                                    preferred_element_type=jnp.float32)
        k_scr[...] = _rope(k_lin, cos, sin)
        v_scr[...] = jax.lax.dot_general(h, wv_ref[...], _DN,
                                         preferred_element_type=jnp.float32)

    q_lin = jax.lax.dot_general(h_ref[...], wq_ref[...], _DN,
                                preferred_element_type=jnp.float32)
    q = _rope(q_lin, cos, sin) * scaling
    k = k_scr[...]
    v = v_scr[...]

    for i in range(S // BQ):
        kv_len = (i + 1) * BQ
        qi = q[i * BQ:(i + 1) * BQ]
        s = jax.lax.dot_general(qi, k[:kv_len], _DN,
                                preferred_element_type=jnp.float32)
        rows = jax.lax.broadcasted_iota(jnp.int32, (BQ, kv_len), 0) + i * BQ
        cols = jax.lax.broadcasted_iota(jnp.int32, (BQ, kv_len), 1)
        s = jnp.where(rows >= cols, s, -1e30)
        m = jnp.max(s, axis=1, keepdims=True)
        e = jnp.exp(s - m)
        denom = jnp.sum(e, axis=1, keepdims=True)
        oi = jnp.dot(e, v[:kv_len], preferred_element_type=jnp.float32)
        o_ref[0, i * BQ:(i + 1) * BQ, :] = oi / denom


def kernel(hidden_states, cos, sin, Wq, Wk, Wv):
    h2d = hidden_states[0]
    cos2d = cos[0]
    sin2d = sin[0]

    out = pl.pallas_call(
        _fused_kernel,
        grid=(NQ,),
        in_specs=[
            pl.BlockSpec((S, HIDDEN), lambda h: (0, 0)),
            pl.BlockSpec((HD, HIDDEN), lambda h: (h, 0)),
            pl.BlockSpec((HD, HIDDEN), lambda h: (h // GROUP, 0)),
            pl.BlockSpec((HD, HIDDEN), lambda h: (h // GROUP, 0)),
            pl.BlockSpec((S, HD), lambda h: (0, 0)),
            pl.BlockSpec((S, HD), lambda h: (0, 0)),
        ],
        out_specs=pl.BlockSpec((1, S, HD), lambda h: (h, 0, 0)),
        out_shape=jax.ShapeDtypeStruct((NQ, S, HD), jnp.float32),
        scratch_shapes=[
            pltpu.VMEM((S, HD), jnp.float32),
            pltpu.VMEM((S, HD), jnp.float32),
        ],
    )(h2d, Wq, Wk, Wv, cos2d, sin2d)
    return out


# speedup vs baseline: 2.0241x; 1.5956x over previous
import jax
import jax.numpy as jnp
from jax.experimental import pallas as pl

HIDDEN = 1024
NQ = 16
NKV = 4
HD = 64
S = 1024
GROUP = NQ // NKV
BQ = 256

_DN = (((1,), (1,)), ((), ()))


def _rope_full(x, cos_t, sin_t):
    w = x.shape[1]
    r_minus = jnp.roll(x, -HD // 2, axis=1)
    r_plus = jnp.roll(x, HD // 2, axis=1)
    lane = jax.lax.broadcasted_iota(jnp.int32, x.shape, 1)
    rot = jnp.where(lane % HD < HD // 2, -r_minus, r_plus)
    return x * cos_t, rot * sin_t, x * cos_t + rot * sin_t


def _fused_kernel(h_ref, wq_ref, wk_ref, wv_ref, cos_ref, sin_ref, o_ref):
    scaling = HD ** (-0.5)
    h = h_ref[...]
    cos = cos_ref[...]
    sin = sin_ref[...]

    q_lin = jax.lax.dot_general(h, wq_ref[...], _DN,
                                preferred_element_type=jnp.float32)
    k_lin = jax.lax.dot_general(h, wk_ref[...], _DN,
                                preferred_element_type=jnp.float32)
    v = jax.lax.dot_general(h, wv_ref[...], _DN,
                            preferred_element_type=jnp.float32)

    cos_q = jnp.tile(cos, (1, NQ))
    sin_q = jnp.tile(sin, (1, NQ))
    cos_k = jnp.tile(cos, (1, NKV))
    sin_k = jnp.tile(sin, (1, NKV))
    *_, q = _rope_full(q_lin, cos_q, sin_q)
    q = q * scaling
    *_, k = _rope_full(k_lin, cos_k, sin_k)

    ones = jnp.ones((S, HD), dtype=jnp.float32)

    for g in range(NKV):
        k_g = k[:, g * HD:(g + 1) * HD]
        v_aug = jnp.concatenate([v[:, g * HD:(g + 1) * HD], ones], axis=1)
        for hh in range(GROUP):
            head = g * GROUP + hh
            q_h = q[:, head * HD:(head + 1) * HD]
            for i in range(S // BQ):
                kv_len = (i + 1) * BQ
                qi = q_h[i * BQ:(i + 1) * BQ]
                s = jax.lax.dot_general(qi, k_g[:kv_len], _DN,
                                        preferred_element_type=jnp.float32)
                rows = jax.lax.broadcasted_iota(
                    jnp.int32, (BQ, kv_len), 0) + i * BQ
                cols = jax.lax.broadcasted_iota(jnp.int32, (BQ, kv_len), 1)
                s = jnp.where(rows >= cols, s, -1e30)
                m = jnp.max(s, axis=1, keepdims=True)
                e = jnp.exp(s - m)
                o_aug = jnp.dot(e, v_aug[:kv_len],
                                preferred_element_type=jnp.float32)
                o_ref[head, i * BQ:(i + 1) * BQ, :] = (
                    o_aug[:, :HD] / o_aug[:, HD:])


def kernel(hidden_states, cos, sin, Wq, Wk, Wv):
    h2d = hidden_states[0]
    cos2d = cos[0]
    sin2d = sin[0]

    out = pl.pallas_call(
        _fused_kernel,
        grid=(1,),
        in_specs=[
            pl.BlockSpec((S, HIDDEN), lambda i: (0, 0)),
            pl.BlockSpec((NQ * HD, HIDDEN), lambda i: (0, 0)),
            pl.BlockSpec((NKV * HD, HIDDEN), lambda i: (0, 0)),
            pl.BlockSpec((NKV * HD, HIDDEN), lambda i: (0, 0)),
            pl.BlockSpec((S, HD), lambda i: (0, 0)),
            pl.BlockSpec((S, HD), lambda i: (0, 0)),
        ],
        out_specs=pl.BlockSpec((NQ, S, HD), lambda i: (0, 0, 0)),
        out_shape=jax.ShapeDtypeStruct((NQ, S, HD), jnp.float32),
    )(h2d, Wq, Wk, Wv, cos2d, sin2d)
    return out
